# Initial kernel scaffold; baseline (speedup 1.0000x reference)
import jax
import jax.numpy as jnp
from jax.experimental import pallas as pl


def kernel(query, reference_pts, value_src, spatial_shape, W_off, b_off, W_attn, b_attn, W_val, b_val, W_out, b_out):
    raise NotImplementedError("write your pallas kernel here")


# speedup vs baseline: 2663.4122x; 2663.4122x over previous
import functools

import numpy as np
import jax
import jax.numpy as jnp
from jax import lax
from jax.experimental import pallas as pl
from jax.experimental.pallas import tpu as pltpu
from jax.experimental.pallas import tpu_sc as plsc

_B = 4
_H = 64
_W = 64
_N = _H * _W
_D = 256
_NH = 8
_NP = 4
_DH = _D // _NH
_NTAP = _NP * 4
_NC = 128

_NBLK = 512
_CQ = 8
_SC_CORES = 2
_SC_SUBCORES = 16
_NWORKERS = _SC_CORES * _SC_SUBCORES
_QPW = (_B * _N) // _NWORKERS


def _prep_body(q_ref, r_ref, vs_ref, wval_ref, bval_ref, woff_ref, boff_ref,
               wattn_ref, battn_ref, val_ref, idx_ref, wgt_ref):
    b = pl.program_id(0)
    q = q_ref[0]
    val_ref[0] = (jnp.dot(vs_ref[0], wval_ref[...],
                          preferred_element_type=jnp.float32) + bval_ref[...])
    offs = (jnp.dot(q, woff_ref[...], preferred_element_type=jnp.float32)
            + boff_ref[...])
    attn = (jnp.dot(q, wattn_ref[...], preferred_element_type=jnp.float32)
            + battn_ref[...])
    m = jnp.maximum(jnp.maximum(attn[:, 0:8], attn[:, 8:16]),
                    jnp.maximum(attn[:, 16:24], attn[:, 24:32]))
    ex = jnp.exp(attn - jnp.concatenate([m, m, m, m], axis=1))
    s = ex[:, 0:8] + ex[:, 8:16] + ex[:, 16:24] + ex[:, 24:32]
    att = ex / jnp.concatenate([s, s, s, s], axis=1)

    rx = r_ref[0][:, 0:1]
    ry = r_ref[0][:, 1:2]
    lx = jnp.clip(rx + offs[:, 0:32] * (1.0 / _W), 0.0, 1.0)
    ly = jnp.clip(ry + offs[:, 32:64] * (1.0 / _H), 0.0, 1.0)
    gx = lx * 2.0 - 1.0
    gy = ly * 2.0 - 1.0
    x = (gx + 1.0) * (_W / 2.0) - 0.5
    y = (gy + 1.0) * (_H / 2.0) - 0.5
    x0 = jnp.floor(x)
    y0 = jnp.floor(y)
    wx1 = x - x0
    wx0 = 1.0 - wx1
    wy1 = y - y0
    wy0 = 1.0 - wy1

    hcol = lax.broadcasted_iota(jnp.int32, (_NBLK, 32), 1) % _NH
    base = b * (_N * _NH)

    def corner(xi, yi, wxy):
        valid = ((xi >= 0.0) & (xi <= _W - 1.0)
                 & (yi >= 0.0) & (yi <= _H - 1.0))
        xc = jnp.clip(xi, 0.0, _W - 1.0).astype(jnp.int32)
        yc = jnp.clip(yi, 0.0, _H - 1.0).astype(jnp.int32)
        gidx = base + (yc * _W + xc) * _NH + hcol
        w = att * wxy * valid.astype(jnp.float32)
        return gidx, w

    i00, w00 = corner(x0, y0, wx0 * wy0)
    i01, w01 = corner(x0 + 1.0, y0, wx1 * wy0)
    i10, w10 = corner(x0, y0 + 1.0, wx0 * wy1)
    i11, w11 = corner(x0 + 1.0, y0 + 1.0, wx1 * wy1)
    idx_ref[0] = jnp.concatenate([i00, i01, i10, i11], axis=1)
    wgt_ref[0] = jnp.concatenate([w00, w01, w10, w11], axis=1)


def _prep(query, reference_pts, value_src, W_val, b_val, W_off_p, b_off_p,
          W_attn_p, b_attn_p):
    grid = (_B, _N // _NBLK)
    return pl.pallas_call(
        _prep_body,
        grid=grid,
        in_specs=[
            pl.BlockSpec((1, _NBLK, _D), lambda b, i: (b, i, 0)),
            pl.BlockSpec((1, _NBLK, 2), lambda b, i: (b, i, 0)),
            pl.BlockSpec((1, _NBLK, _D), lambda b, i: (b, i, 0)),
            pl.BlockSpec((_D, _D), lambda b, i: (0, 0)),
            pl.BlockSpec((1, _D), lambda b, i: (0, 0)),
            pl.BlockSpec((_D, 64), lambda b, i: (0, 0)),
            pl.BlockSpec((1, 64), lambda b, i: (0, 0)),
            pl.BlockSpec((_D, 32), lambda b, i: (0, 0)),
            pl.BlockSpec((1, 32), lambda b, i: (0, 0)),
        ],
        out_specs=[
            pl.BlockSpec((1, _NBLK, _D), lambda b, i: (b, i, 0)),
            pl.BlockSpec((1, _NBLK, _NC), lambda b, i: (b, i, 0)),
            pl.BlockSpec((1, _NBLK, _NC), lambda b, i: (b, i, 0)),
        ],
        out_shape=[
            jax.ShapeDtypeStruct((_B, _N, _D), jnp.float32),
            jax.ShapeDtypeStruct((_B, _N, _NC), jnp.int32),
            jax.ShapeDtypeStruct((_B, _N, _NC), jnp.float32),
        ],
    )(query, reference_pts, value_src, W_val, b_val, W_off_p, b_off_p,
      W_attn_p, b_attn_p)


def _sc_body(table_hbm, idx_hbm, wgt_hbm, out_hbm, idx_v, wgt_v, rows_v,
             out_v, sem):
    wid = lax.axis_index("s") * _SC_CORES + lax.axis_index("c")
    qbase = wid * _QPW

    def chunk(g, carry):
        q0 = qbase + g * _CQ
        pltpu.sync_copy(idx_hbm.at[pl.ds(q0, _CQ)], idx_v)
        pltpu.sync_copy(wgt_hbm.at[pl.ds(q0, _CQ)], wgt_v)
        copies = [pltpu.async_copy(table_hbm.at[idx_v.at[q]], rows_v.at[q], sem)
                  for q in range(_CQ)]
        for c in copies:
            c.wait()

        def qloop(q, c2):
            wv = [wgt_v[q, pl.ds(k * 16, 16)] for k in range(_NC // 16)]
            for h in range(_NH):
                a0 = jnp.zeros((16,), jnp.float32)
                a1 = jnp.zeros((16,), jnp.float32)
                for cc in range(4):
                    for p in range(_NP):
                        j = cc * 32 + p * 8 + h
                        w = wv[j // 16][j % 16]
                        a0 = a0 + w * rows_v[q, j, pl.ds(0, 16)]
                        a1 = a1 + w * rows_v[q, j, pl.ds(16, 16)]
                out_v[q * _NH + h, pl.ds(0, 16)] = a0
                out_v[q * _NH + h, pl.ds(16, 16)] = a1
            return c2

        lax.fori_loop(0, _CQ, qloop, 0)
        pltpu.sync_copy(out_v, out_hbm.at[pl.ds(q0 * _NH, _CQ * _NH)])
        return carry

    lax.fori_loop(0, _QPW // _CQ, chunk, 0)


@functools.lru_cache(maxsize=1)
def _sc_gather_combine():
    return pl.kernel(
        _sc_body,
        out_type=jax.ShapeDtypeStruct((_B * _N * _NH, _DH), jnp.float32),
        mesh=plsc.VectorSubcoreMesh(core_axis_name="c", subcore_axis_name="s"),
        compiler_params=pltpu.CompilerParams(use_tc_tiling_on_sc=False),
        scratch_types=[
            pltpu.VMEM((_CQ, _NC), jnp.int32),
            pltpu.VMEM((_CQ, _NC), jnp.float32),
            pltpu.VMEM((_CQ, _NC, _DH), jnp.float32),
            pltpu.VMEM((_CQ * _NH, _DH), jnp.float32),
            pltpu.SemaphoreType.DMA,
        ],
    )


def _proj_body(x_ref, w_ref, b_ref, o_ref):
    o_ref[0] = (jnp.dot(x_ref[0], w_ref[...],
                        preferred_element_type=jnp.float32) + b_ref[...])


def _out_proj(x, W_out, b_out):
    grid = (_B, _N // _NBLK)
    return pl.pallas_call(
        _proj_body,
        grid=grid,
        in_specs=[
            pl.BlockSpec((1, _NBLK, _D), lambda b, i: (b, i, 0)),
            pl.BlockSpec((_D, _D), lambda b, i: (0, 0)),
            pl.BlockSpec((1, _D), lambda b, i: (0, 0)),
        ],
        out_specs=pl.BlockSpec((1, _NBLK, _D), lambda b, i: (b, i, 0)),
        out_shape=jax.ShapeDtypeStruct((_B, _N, _D), jnp.float32),
    )(x, W_out, b_out)


def _perms():
    perm_off = np.empty(64, np.int32)
    for k in range(2):
        for p in range(_NP):
            for h in range(_NH):
                perm_off[k * 32 + p * 8 + h] = (h * _NP + p) * 2 + k
    perm_attn = np.empty(32, np.int32)
    for p in range(_NP):
        for h in range(_NH):
            perm_attn[p * 8 + h] = h * _NP + p
    return perm_off, perm_attn


_PERM_OFF, _PERM_ATTN = _perms()


def kernel(query, reference_pts, value_src, spatial_shape, W_off, b_off,
           W_attn, b_attn, W_val, b_val, W_out, b_out):
    del spatial_shape
    W_off_p = W_off[:, _PERM_OFF]
    b_off_p = b_off[_PERM_OFF].reshape(1, 64)
    W_attn_p = W_attn[:, _PERM_ATTN]
    b_attn_p = b_attn[_PERM_ATTN].reshape(1, 32)

    value, idx, wgt = _prep(query, reference_pts, value_src, W_val,
                            b_val.reshape(1, _D), W_off_p, b_off_p,
                            W_attn_p, b_attn_p)
    table = value.reshape(_B * _N * _NH, _DH)
    rows = _sc_gather_combine()(table, idx.reshape(_B * _N, _NC),
                                wgt.reshape(_B * _N, _NC))
    return _out_proj(rows.reshape(_B, _N, _D), W_out, b_out.reshape(1, _D))


# speedup vs baseline: 3575.1188x; 1.3423x over previous
import functools

import numpy as np
import jax
import jax.numpy as jnp
from jax import lax
from jax.experimental import pallas as pl
from jax.experimental.pallas import tpu as pltpu
from jax.experimental.pallas import tpu_sc as plsc

_B = 4
_H = 64
_W = 64
_N = _H * _W
_D = 256
_NH = 8
_NP = 4
_DH = _D // _NH
_NTAP = _NP * 4
_NC = 128

_NBLK = 512
_CQ = 8
_SC_CORES = 2
_SC_SUBCORES = 16
_NWORKERS = _SC_CORES * _SC_SUBCORES
_QPW = (_B * _N) // _NWORKERS


def _prep_body(q_ref, r_ref, vs_ref, wval_ref, bval_ref, woff_ref, boff_ref,
               wattn_ref, battn_ref, val_ref, idx_ref, wgt_ref):
    b = pl.program_id(0)
    q = q_ref[0]
    val_ref[0] = (jnp.dot(vs_ref[0], wval_ref[...],
                          preferred_element_type=jnp.float32) + bval_ref[...])
    offs = (jnp.dot(q, woff_ref[...], preferred_element_type=jnp.float32)
            + boff_ref[...])
    attn = (jnp.dot(q, wattn_ref[...], preferred_element_type=jnp.float32)
            + battn_ref[...])
    m = jnp.maximum(jnp.maximum(attn[:, 0:8], attn[:, 8:16]),
                    jnp.maximum(attn[:, 16:24], attn[:, 24:32]))
    ex = jnp.exp(attn - jnp.concatenate([m, m, m, m], axis=1))
    s = ex[:, 0:8] + ex[:, 8:16] + ex[:, 16:24] + ex[:, 24:32]
    att = ex / jnp.concatenate([s, s, s, s], axis=1)

    rx = r_ref[0][:, 0:1]
    ry = r_ref[0][:, 1:2]
    lx = jnp.clip(rx + offs[:, 0:32] * (1.0 / _W), 0.0, 1.0)
    ly = jnp.clip(ry + offs[:, 32:64] * (1.0 / _H), 0.0, 1.0)
    gx = lx * 2.0 - 1.0
    gy = ly * 2.0 - 1.0
    x = (gx + 1.0) * (_W / 2.0) - 0.5
    y = (gy + 1.0) * (_H / 2.0) - 0.5
    x0 = jnp.floor(x)
    y0 = jnp.floor(y)
    wx1 = x - x0
    wx0 = 1.0 - wx1
    wy1 = y - y0
    wy0 = 1.0 - wy1

    hcol = lax.broadcasted_iota(jnp.int32, (_NBLK, 32), 1) % _NH
    base = b * (_N * _NH)

    def corner(xi, yi, wxy):
        valid = ((xi >= 0.0) & (xi <= _W - 1.0)
                 & (yi >= 0.0) & (yi <= _H - 1.0))
        xc = jnp.clip(xi, 0.0, _W - 1.0).astype(jnp.int32)
        yc = jnp.clip(yi, 0.0, _H - 1.0).astype(jnp.int32)
        gidx = base + (yc * _W + xc) * _NH + hcol
        w = att * wxy * valid.astype(jnp.float32)
        return gidx, w

    i00, w00 = corner(x0, y0, wx0 * wy0)
    i01, w01 = corner(x0 + 1.0, y0, wx1 * wy0)
    i10, w10 = corner(x0, y0 + 1.0, wx0 * wy1)
    i11, w11 = corner(x0 + 1.0, y0 + 1.0, wx1 * wy1)
    idx_ref[0] = jnp.concatenate([i00, i01, i10, i11], axis=1)
    wgt_ref[0] = jnp.concatenate([w00, w01, w10, w11], axis=1)


def _prep(query, reference_pts, value_src, W_val, b_val, W_off_p, b_off_p,
          W_attn_p, b_attn_p):
    grid = (_B, _N // _NBLK)
    return pl.pallas_call(
        _prep_body,
        grid=grid,
        in_specs=[
            pl.BlockSpec((1, _NBLK, _D), lambda b, i: (b, i, 0)),
            pl.BlockSpec((1, _NBLK, 2), lambda b, i: (b, i, 0)),
            pl.BlockSpec((1, _NBLK, _D), lambda b, i: (b, i, 0)),
            pl.BlockSpec((_D, _D), lambda b, i: (0, 0)),
            pl.BlockSpec((1, _D), lambda b, i: (0, 0)),
            pl.BlockSpec((_D, 64), lambda b, i: (0, 0)),
            pl.BlockSpec((1, 64), lambda b, i: (0, 0)),
            pl.BlockSpec((_D, 32), lambda b, i: (0, 0)),
            pl.BlockSpec((1, 32), lambda b, i: (0, 0)),
        ],
        out_specs=[
            pl.BlockSpec((1, _NBLK, _D), lambda b, i: (b, i, 0)),
            pl.BlockSpec((1, _NBLK, _NC), lambda b, i: (b, i, 0)),
            pl.BlockSpec((1, _NBLK, _NC), lambda b, i: (b, i, 0)),
        ],
        out_shape=[
            jax.ShapeDtypeStruct((_B, _N, _D), jnp.float32),
            jax.ShapeDtypeStruct((_B, _N, _NC), jnp.int32),
            jax.ShapeDtypeStruct((_B, _N, _NC), jnp.float32),
        ],
    )(query, reference_pts, value_src, W_val, b_val, W_off_p, b_off_p,
      W_attn_p, b_attn_p)


def _sc_body(table_hbm, idx_hbm, wgt_hbm, out_hbm, idx_v, wgt_v, rows_v,
             out_v, sem0, sem1):
    wid = lax.axis_index("s") * _SC_CORES + lax.axis_index("c")
    qbase = wid * _QPW
    nchunks = _QPW // _CQ
    sems = (sem0, sem1)

    def prefetch(slot, g):
        q0 = qbase + g * _CQ
        pltpu.sync_copy(idx_hbm.at[pl.ds(q0, _CQ)], idx_v.at[slot])
        pltpu.sync_copy(wgt_hbm.at[pl.ds(q0, _CQ)], wgt_v.at[slot])
        for q in range(_CQ):
            pltpu.async_copy(table_hbm.at[idx_v.at[slot, q]],
                             rows_v.at[slot, q], sems[slot])

    def process(slot, g):
        q0 = qbase + g * _CQ
        for q in range(_CQ):
            pltpu.make_async_copy(table_hbm.at[idx_v.at[slot, q]],
                                  rows_v.at[slot, q], sems[slot]).wait()

        def qloop(q, c2):
            wv = [wgt_v[slot, q, pl.ds(k * 16, 16)] for k in range(_NC // 16)]
            for h in range(_NH):
                a0 = jnp.zeros((16,), jnp.float32)
                a1 = jnp.zeros((16,), jnp.float32)
                for cc in range(4):
                    for p in range(_NP):
                        j = cc * 32 + p * 8 + h
                        w = wv[j // 16][j % 16]
                        a0 = a0 + w * rows_v[slot, q, j, pl.ds(0, 16)]
                        a1 = a1 + w * rows_v[slot, q, j, pl.ds(16, 16)]
                out_v[q * _NH + h, pl.ds(0, 16)] = a0
                out_v[q * _NH + h, pl.ds(16, 16)] = a1
            return c2

        lax.fori_loop(0, _CQ, qloop, 0)
        pltpu.sync_copy(out_v, out_hbm.at[pl.ds(q0 * _NH, _CQ * _NH)])

    prefetch(0, 0)

    def pair(gg, carry):
        g = gg * 2
        prefetch(1, g + 1)
        process(0, g)

        @pl.when(g + 2 < nchunks)
        def _():
            prefetch(0, g + 2)

        process(1, g + 1)
        return carry

    lax.fori_loop(0, nchunks // 2, pair, 0)


@functools.lru_cache(maxsize=1)
def _sc_gather_combine():
    return pl.kernel(
        _sc_body,
        out_type=jax.ShapeDtypeStruct((_B * _N * _NH, _DH), jnp.float32),
        mesh=plsc.VectorSubcoreMesh(core_axis_name="c", subcore_axis_name="s"),
        compiler_params=pltpu.CompilerParams(use_tc_tiling_on_sc=False),
        scratch_types=[
            pltpu.VMEM((2, _CQ, _NC), jnp.int32),
            pltpu.VMEM((2, _CQ, _NC), jnp.float32),
            pltpu.VMEM((2, _CQ, _NC, _DH), jnp.float32),
            pltpu.VMEM((_CQ * _NH, _DH), jnp.float32),
            pltpu.SemaphoreType.DMA,
            pltpu.SemaphoreType.DMA,
        ],
    )


def _proj_body(x_ref, w_ref, b_ref, o_ref):
    o_ref[0] = (jnp.dot(x_ref[0], w_ref[...],
                        preferred_element_type=jnp.float32) + b_ref[...])


def _out_proj(x, W_out, b_out):
    grid = (_B, _N // _NBLK)
    return pl.pallas_call(
        _proj_body,
        grid=grid,
        in_specs=[
            pl.BlockSpec((1, _NBLK, _D), lambda b, i: (b, i, 0)),
            pl.BlockSpec((_D, _D), lambda b, i: (0, 0)),
            pl.BlockSpec((1, _D), lambda b, i: (0, 0)),
        ],
        out_specs=pl.BlockSpec((1, _NBLK, _D), lambda b, i: (b, i, 0)),
        out_shape=jax.ShapeDtypeStruct((_B, _N, _D), jnp.float32),
    )(x, W_out, b_out)


def _perms():
    perm_off = np.empty(64, np.int32)
    for k in range(2):
        for p in range(_NP):
            for h in range(_NH):
                perm_off[k * 32 + p * 8 + h] = (h * _NP + p) * 2 + k
    perm_attn = np.empty(32, np.int32)
    for p in range(_NP):
        for h in range(_NH):
            perm_attn[p * 8 + h] = h * _NP + p
    return perm_off, perm_attn


_PERM_OFF, _PERM_ATTN = _perms()


def kernel(query, reference_pts, value_src, spatial_shape, W_off, b_off,
           W_attn, b_attn, W_val, b_val, W_out, b_out):
    del spatial_shape
    W_off_p = W_off[:, _PERM_OFF]
    b_off_p = b_off[_PERM_OFF].reshape(1, 64)
    W_attn_p = W_attn[:, _PERM_ATTN]
    b_attn_p = b_attn[_PERM_ATTN].reshape(1, 32)

    value, idx, wgt = _prep(query, reference_pts, value_src, W_val,
                            b_val.reshape(1, _D), W_off_p, b_off_p,
                            W_attn_p, b_attn_p)
    table = value.reshape(_B * _N * _NH, _DH)
    rows = _sc_gather_combine()(table, idx.reshape(_B * _N, _NC),
                                wgt.reshape(_B * _N, _NC))
    return _out_proj(rows.reshape(_B, _N, _D), W_out, b_out.reshape(1, _D))


# speedup vs baseline: 3609.8249x; 1.0097x over previous
import functools

import numpy as np
import jax
import jax.numpy as jnp
from jax import lax
from jax.experimental import pallas as pl
from jax.experimental.pallas import tpu as pltpu
from jax.experimental.pallas import tpu_sc as plsc

_B = 4
_H = 64
_W = 64
_N = _H * _W
_D = 256
_NH = 8
_NP = 4
_DH = _D // _NH
_NTAP = _NP * 4
_NC = 128

_NBLK = 512
_CQ = 8
_SC_CORES = 2
_SC_SUBCORES = 16
_NWORKERS = _SC_CORES * _SC_SUBCORES
_QPW = (_B * _N) // _NWORKERS


def _prep_body(q_ref, r_ref, vs_ref, wval_ref, bval_ref, woff_ref, boff_ref,
               wattn_ref, battn_ref, val_ref, idx_ref, wgt_ref):
    b = pl.program_id(0)
    q = q_ref[0]
    val_ref[0] = (jnp.dot(vs_ref[0], wval_ref[...],
                          preferred_element_type=jnp.float32)
                  + bval_ref[...]).astype(jnp.bfloat16)
    offs = (jnp.dot(q, woff_ref[...], preferred_element_type=jnp.float32)
            + boff_ref[...])
    attn = (jnp.dot(q, wattn_ref[...], preferred_element_type=jnp.float32)
            + battn_ref[...])
    m = jnp.maximum(jnp.maximum(attn[:, 0:8], attn[:, 8:16]),
                    jnp.maximum(attn[:, 16:24], attn[:, 24:32]))
    ex = jnp.exp(attn - jnp.concatenate([m, m, m, m], axis=1))
    s = ex[:, 0:8] + ex[:, 8:16] + ex[:, 16:24] + ex[:, 24:32]
    att = ex / jnp.concatenate([s, s, s, s], axis=1)

    rx = r_ref[0][:, 0:1]
    ry = r_ref[0][:, 1:2]
    lx = jnp.clip(rx + offs[:, 0:32] * (1.0 / _W), 0.0, 1.0)
    ly = jnp.clip(ry + offs[:, 32:64] * (1.0 / _H), 0.0, 1.0)
    gx = lx * 2.0 - 1.0
    gy = ly * 2.0 - 1.0
    x = (gx + 1.0) * (_W / 2.0) - 0.5
    y = (gy + 1.0) * (_H / 2.0) - 0.5
    x0 = jnp.floor(x)
    y0 = jnp.floor(y)
    wx1 = x - x0
    wx0 = 1.0 - wx1
    wy1 = y - y0
    wy0 = 1.0 - wy1

    hcol = lax.broadcasted_iota(jnp.int32, (_NBLK, 32), 1) % _NH
    base = b * (_N * _NH)

    def corner(xi, yi, wxy):
        valid = ((xi >= 0.0) & (xi <= _W - 1.0)
                 & (yi >= 0.0) & (yi <= _H - 1.0))
        xc = jnp.clip(xi, 0.0, _W - 1.0).astype(jnp.int32)
        yc = jnp.clip(yi, 0.0, _H - 1.0).astype(jnp.int32)
        gidx = base + (yc * _W + xc) * _NH + hcol
        w = att * wxy * valid.astype(jnp.float32)
        return gidx, w

    i00, w00 = corner(x0, y0, wx0 * wy0)
    i01, w01 = corner(x0 + 1.0, y0, wx1 * wy0)
    i10, w10 = corner(x0, y0 + 1.0, wx0 * wy1)
    i11, w11 = corner(x0 + 1.0, y0 + 1.0, wx1 * wy1)
    idx_ref[0] = jnp.concatenate([i00, i01, i10, i11], axis=1)
    wgt_ref[0] = jnp.concatenate([w00, w01, w10, w11], axis=1)


def _prep(query, reference_pts, value_src, W_val, b_val, W_off_p, b_off_p,
          W_attn_p, b_attn_p):
    grid = (_B, _N // _NBLK)
    return pl.pallas_call(
        _prep_body,
        grid=grid,
        in_specs=[
            pl.BlockSpec((1, _NBLK, _D), lambda b, i: (b, i, 0)),
            pl.BlockSpec((1, _NBLK, 2), lambda b, i: (b, i, 0)),
            pl.BlockSpec((1, _NBLK, _D), lambda b, i: (b, i, 0)),
            pl.BlockSpec((_D, _D), lambda b, i: (0, 0)),
            pl.BlockSpec((1, _D), lambda b, i: (0, 0)),
            pl.BlockSpec((_D, 64), lambda b, i: (0, 0)),
            pl.BlockSpec((1, 64), lambda b, i: (0, 0)),
            pl.BlockSpec((_D, 32), lambda b, i: (0, 0)),
            pl.BlockSpec((1, 32), lambda b, i: (0, 0)),
        ],
        out_specs=[
            pl.BlockSpec((1, _NBLK, _D), lambda b, i: (b, i, 0)),
            pl.BlockSpec((1, _NBLK, _NC), lambda b, i: (b, i, 0)),
            pl.BlockSpec((1, _NBLK, _NC), lambda b, i: (b, i, 0)),
        ],
        out_shape=[
            jax.ShapeDtypeStruct((_B, _N, _D), jnp.bfloat16),
            jax.ShapeDtypeStruct((_B, _N, _NC), jnp.int32),
            jax.ShapeDtypeStruct((_B, _N, _NC), jnp.float32),
        ],
    )(query, reference_pts, value_src, W_val, b_val, W_off_p, b_off_p,
      W_attn_p, b_attn_p)


def _sc_body(table_hbm, idx_hbm, wgt_hbm, out_hbm, idx_v, wgt_v, rows_v,
             out_v, sem0, sem1):
    wid = lax.axis_index("s") * _SC_CORES + lax.axis_index("c")
    qbase = wid * _QPW
    nchunks = _QPW // _CQ
    sems = (sem0, sem1)

    def prefetch(slot, g):
        q0 = qbase + g * _CQ
        pltpu.sync_copy(idx_hbm.at[pl.ds(q0, _CQ)], idx_v.at[slot])
        pltpu.sync_copy(wgt_hbm.at[pl.ds(q0, _CQ)], wgt_v.at[slot])
        for q in range(_CQ):
            pltpu.async_copy(table_hbm.at[idx_v.at[slot, q]],
                             rows_v.at[slot, q], sems[slot])

    def process(slot, g):
        q0 = qbase + g * _CQ
        for q in range(_CQ):
            pltpu.make_async_copy(table_hbm.at[idx_v.at[slot, q]],
                                  rows_v.at[slot, q], sems[slot]).wait()

        def qloop(q, c2):
            wv = [wgt_v[slot, q, pl.ds(k * 16, 16)] for k in range(_NC // 16)]
            for h in range(_NH):
                a0 = jnp.zeros((16,), jnp.float32)
                a1 = jnp.zeros((16,), jnp.float32)
                for cc in range(4):
                    for p in range(_NP):
                        j = cc * 32 + p * 8 + h
                        w = wv[j // 16][j % 16]
                        e, o = plsc.unpack(rows_v[slot, q, j, :],
                                           format=plsc.PackFormat.INTERLEAVED,
                                           preferred_element_type=jnp.float32)
                        a0 = a0 + w * e
                        a1 = a1 + w * o
                out_v[q * _NH + h, pl.ds(0, 16)] = a0
                out_v[q * _NH + h, pl.ds(16, 16)] = a1
            return c2

        lax.fori_loop(0, _CQ, qloop, 0)
        pltpu.sync_copy(out_v, out_hbm.at[pl.ds(q0 * _NH, _CQ * _NH)])

    prefetch(0, 0)

    def pair(gg, carry):
        g = gg * 2
        prefetch(1, g + 1)
        process(0, g)

        @pl.when(g + 2 < nchunks)
        def _():
            prefetch(0, g + 2)

        process(1, g + 1)
        return carry

    lax.fori_loop(0, nchunks // 2, pair, 0)


@functools.lru_cache(maxsize=1)
def _sc_gather_combine():
    return pl.kernel(
        _sc_body,
        out_type=jax.ShapeDtypeStruct((_B * _N * _NH, _DH), jnp.float32),
        mesh=plsc.VectorSubcoreMesh(core_axis_name="c", subcore_axis_name="s"),
        compiler_params=pltpu.CompilerParams(use_tc_tiling_on_sc=False,
                                             needs_layout_passes=False),
        scratch_types=[
            pltpu.VMEM((2, _CQ, _NC), jnp.int32),
            pltpu.VMEM((2, _CQ, _NC), jnp.float32),
            pltpu.VMEM((2, _CQ, _NC, _DH), jnp.bfloat16),
            pltpu.VMEM((_CQ * _NH, _DH), jnp.float32),
            pltpu.SemaphoreType.DMA,
            pltpu.SemaphoreType.DMA,
        ],
    )


def _proj_body(x_ref, w_ref, b_ref, o_ref):
    o_ref[0] = (jnp.dot(x_ref[0], w_ref[...],
                        preferred_element_type=jnp.float32) + b_ref[...])


def _out_proj(x, W_out, b_out):
    grid = (_B, _N // _NBLK)
    return pl.pallas_call(
        _proj_body,
        grid=grid,
        in_specs=[
            pl.BlockSpec((1, _NBLK, _D), lambda b, i: (b, i, 0)),
            pl.BlockSpec((_D, _D), lambda b, i: (0, 0)),
            pl.BlockSpec((1, _D), lambda b, i: (0, 0)),
        ],
        out_specs=pl.BlockSpec((1, _NBLK, _D), lambda b, i: (b, i, 0)),
        out_shape=jax.ShapeDtypeStruct((_B, _N, _D), jnp.float32),
    )(x, W_out, b_out)


def _perms():
    perm_off = np.empty(64, np.int32)
    for k in range(2):
        for p in range(_NP):
            for h in range(_NH):
                perm_off[k * 32 + p * 8 + h] = (h * _NP + p) * 2 + k
    perm_attn = np.empty(32, np.int32)
    for p in range(_NP):
        for h in range(_NH):
            perm_attn[p * 8 + h] = h * _NP + p
    perm_out = np.empty(_D, np.int32)
    for h in range(_NH):
        for i in range(_DH):
            ch = 2 * i if i < 16 else 2 * (i - 16) + 1
            perm_out[h * _DH + i] = h * _DH + ch
    return perm_off, perm_attn, perm_out


_PERM_OFF, _PERM_ATTN, _PERM_OUT = _perms()


def kernel(query, reference_pts, value_src, spatial_shape, W_off, b_off,
           W_attn, b_attn, W_val, b_val, W_out, b_out):
    del spatial_shape
    W_off_p = W_off[:, _PERM_OFF]
    b_off_p = b_off[_PERM_OFF].reshape(1, 64)
    W_attn_p = W_attn[:, _PERM_ATTN]
    b_attn_p = b_attn[_PERM_ATTN].reshape(1, 32)

    value, idx, wgt = _prep(query, reference_pts, value_src, W_val,
                            b_val.reshape(1, _D), W_off_p, b_off_p,
                            W_attn_p, b_attn_p)
    table = value.reshape(_B * _N * _NH, _DH)
    rows = _sc_gather_combine()(table, idx.reshape(_B * _N, _NC),
                                wgt.reshape(_B * _N, _NC))
    return _out_proj(rows.reshape(_B, _N, _D), W_out[_PERM_OUT, :],
                     b_out.reshape(1, _D))


# speedup vs baseline: 4370.7057x; 1.2108x over previous
import functools

import numpy as np
import jax
import jax.numpy as jnp
from jax import lax
from jax.experimental import pallas as pl
from jax.experimental.pallas import tpu as pltpu
from jax.experimental.pallas import tpu_sc as plsc

_B = 4
_H = 64
_W = 64
_N = _H * _W
_D = 256
_NH = 8
_NP = 4
_DH = _D // _NH
_NTAP = _NP * 4
_NC = 128

_NBLK = 512
_CQ = 8
_SC_CORES = 2
_SC_SUBCORES = 16
_NWORKERS = _SC_CORES * _SC_SUBCORES
_QPW = (_B * _N) // _NWORKERS


def _prep_body(q_ref, r_ref, vs_ref, wval_ref, bval_ref, woff_ref, boff_ref,
               wattn_ref, battn_ref, val_ref, idx_ref, wgt_ref):
    b = pl.program_id(0)
    q = q_ref[0]
    val_ref[0] = (jnp.dot(vs_ref[0], wval_ref[...],
                          preferred_element_type=jnp.float32)
                  + bval_ref[...]).astype(jnp.bfloat16)
    offs = (jnp.dot(q, woff_ref[...], preferred_element_type=jnp.float32)
            + boff_ref[...])
    attn = (jnp.dot(q, wattn_ref[...], preferred_element_type=jnp.float32)
            + battn_ref[...])
    m = jnp.maximum(jnp.maximum(attn[:, 0:8], attn[:, 8:16]),
                    jnp.maximum(attn[:, 16:24], attn[:, 24:32]))
    ex = jnp.exp(attn - jnp.concatenate([m, m, m, m], axis=1))
    s = ex[:, 0:8] + ex[:, 8:16] + ex[:, 16:24] + ex[:, 24:32]
    att = ex / jnp.concatenate([s, s, s, s], axis=1)

    rx = r_ref[0][:, 0:1]
    ry = r_ref[0][:, 1:2]
    lx = jnp.clip(rx + offs[:, 0:32] * (1.0 / _W), 0.0, 1.0)
    ly = jnp.clip(ry + offs[:, 32:64] * (1.0 / _H), 0.0, 1.0)
    gx = lx * 2.0 - 1.0
    gy = ly * 2.0 - 1.0
    x = (gx + 1.0) * (_W / 2.0) - 0.5
    y = (gy + 1.0) * (_H / 2.0) - 0.5
    x0 = jnp.floor(x)
    y0 = jnp.floor(y)
    wx1 = x - x0
    wx0 = 1.0 - wx1
    wy1 = y - y0
    wy0 = 1.0 - wy1

    hcol = lax.broadcasted_iota(jnp.int32, (_NBLK, 32), 1) % _NH
    base = b * (_N * _NH)

    def corner(xi, yi, wxy):
        valid = ((xi >= 0.0) & (xi <= _W - 1.0)
                 & (yi >= 0.0) & (yi <= _H - 1.0))
        xc = jnp.clip(xi, 0.0, _W - 1.0).astype(jnp.int32)
        yc = jnp.clip(yi, 0.0, _H - 1.0).astype(jnp.int32)
        gidx = base + (yc * _W + xc) * _NH + hcol
        w = att * wxy * valid.astype(jnp.float32)
        return gidx, w

    i00, w00 = corner(x0, y0, wx0 * wy0)
    i01, w01 = corner(x0 + 1.0, y0, wx1 * wy0)
    i10, w10 = corner(x0, y0 + 1.0, wx0 * wy1)
    i11, w11 = corner(x0 + 1.0, y0 + 1.0, wx1 * wy1)
    idx_ref[0] = jnp.concatenate([i00, i01, i10, i11], axis=1)
    wgt_ref[0] = jnp.concatenate([w00, w01, w10, w11], axis=1)


def _prep(query, reference_pts, value_src, W_val, b_val, W_off_p, b_off_p,
          W_attn_p, b_attn_p):
    grid = (_B, _N // _NBLK)
    return pl.pallas_call(
        _prep_body,
        grid=grid,
        in_specs=[
            pl.BlockSpec((1, _NBLK, _D), lambda b, i: (b, i, 0)),
            pl.BlockSpec((1, _NBLK, 2), lambda b, i: (b, i, 0)),
            pl.BlockSpec((1, _NBLK, _D), lambda b, i: (b, i, 0)),
            pl.BlockSpec((_D, _D), lambda b, i: (0, 0)),
            pl.BlockSpec((1, _D), lambda b, i: (0, 0)),
            pl.BlockSpec((_D, 64), lambda b, i: (0, 0)),
            pl.BlockSpec((1, 64), lambda b, i: (0, 0)),
            pl.BlockSpec((_D, 32), lambda b, i: (0, 0)),
            pl.BlockSpec((1, 32), lambda b, i: (0, 0)),
        ],
        out_specs=[
            pl.BlockSpec((1, _NBLK, _D), lambda b, i: (b, i, 0)),
            pl.BlockSpec((1, _NBLK, _NC), lambda b, i: (b, i, 0)),
            pl.BlockSpec((1, _NBLK, _NC), lambda b, i: (b, i, 0)),
        ],
        out_shape=[
            jax.ShapeDtypeStruct((_B, _N, _D), jnp.bfloat16),
            jax.ShapeDtypeStruct((_B, _N, _NC), jnp.int32),
            jax.ShapeDtypeStruct((_B, _N, _NC), jnp.float32),
        ],
    )(query, reference_pts, value_src, W_val, b_val, W_off_p, b_off_p,
      W_attn_p, b_attn_p)


def _sc_body(table_hbm, idx_hbm, wgt_hbm, out_hbm, idx_v, wgt_v, rows_v,
             out_v, isem0, isem1, gsem0, gsem1, osem0, osem1):
    wid = lax.axis_index("s") * _SC_CORES + lax.axis_index("c")
    qbase = wid * _QPW
    nch = _QPW // _CQ
    isems = (isem0, isem1)
    gsems = (gsem0, gsem1)
    osems = (osem0, osem1)

    def fire_idx(slot, g):
        q0 = qbase + g * _CQ
        pltpu.async_copy(idx_hbm.at[pl.ds(q0, _CQ)], idx_v.at[slot],
                         isems[slot])
        pltpu.async_copy(wgt_hbm.at[pl.ds(q0, _CQ)], wgt_v.at[slot],
                         isems[slot])

    def wait_idx(slot):
        pltpu.make_async_copy(idx_hbm.at[pl.ds(0, _CQ)], idx_v.at[slot],
                              isems[slot]).wait()
        pltpu.make_async_copy(wgt_hbm.at[pl.ds(0, _CQ)], wgt_v.at[slot],
                              isems[slot]).wait()

    def fire_gather(slot):
        for q in range(_CQ):
            pltpu.async_copy(table_hbm.at[idx_v.at[slot, q]],
                             rows_v.at[slot, q], gsems[slot])

    def wait_gather(slot):
        for q in range(_CQ):
            pltpu.make_async_copy(table_hbm.at[idx_v.at[slot, q]],
                                  rows_v.at[slot, q], gsems[slot]).wait()

    def fire_out(slot, g):
        q0 = qbase + g * _CQ
        pltpu.async_copy(out_v.at[slot],
                         out_hbm.at[pl.ds(q0 * _NH, _CQ * _NH)], osems[slot])

    def wait_out(slot):
        pltpu.make_async_copy(out_v.at[slot],
                              out_hbm.at[pl.ds(0, _CQ * _NH)],
                              osems[slot]).wait()

    def combine(slot):
        def qloop(q, c2):
            wv = [wgt_v[slot, q, pl.ds(k * 16, 16)] for k in range(_NC // 16)]
            for h in range(_NH):
                a0 = jnp.zeros((16,), jnp.float32)
                a1 = jnp.zeros((16,), jnp.float32)
                for cc in range(4):
                    for p in range(_NP):
                        j = cc * 32 + p * 8 + h
                        w = wv[j // 16][j % 16]
                        e, o = plsc.unpack(rows_v[slot, q, j, :],
                                           format=plsc.PackFormat.INTERLEAVED,
                                           preferred_element_type=jnp.float32)
                        a0 = a0 + w * e
                        a1 = a1 + w * o
                out_v[slot, q * _NH + h, pl.ds(0, 16)] = a0
                out_v[slot, q * _NH + h, pl.ds(16, 16)] = a1
            return c2

        lax.fori_loop(0, _CQ, qloop, 0)

    def iteration(i, s):
        so = 1 - s

        @pl.when(i + 1 < nch)
        def _():
            wait_idx(so)
            fire_gather(so)

        wait_gather(s)

        @pl.when(i + 2 < nch)
        def _():
            fire_idx(s, i + 2)

        @pl.when(i >= 2)
        def _():
            wait_out(s)

        combine(s)
        fire_out(s, i)

    fire_idx(0, 0)
    wait_idx(0)
    fire_gather(0)
    fire_idx(1, 1)

    def pair(gg, carry):
        i = gg * 2
        iteration(i, 0)
        iteration(i + 1, 1)
        return carry

    lax.fori_loop(0, nch // 2, pair, 0)
    wait_out(0)
    wait_out(1)


@functools.lru_cache(maxsize=1)
def _sc_gather_combine():
    return pl.kernel(
        _sc_body,
        out_type=jax.ShapeDtypeStruct((_B * _N * _NH, _DH), jnp.float32),
        mesh=plsc.VectorSubcoreMesh(core_axis_name="c", subcore_axis_name="s"),
        compiler_params=pltpu.CompilerParams(use_tc_tiling_on_sc=False,
                                             needs_layout_passes=False),
        scratch_types=[
            pltpu.VMEM((2, _CQ, _NC), jnp.int32),
            pltpu.VMEM((2, _CQ, _NC), jnp.float32),
            pltpu.VMEM((2, _CQ, _NC, _DH), jnp.bfloat16),
            pltpu.VMEM((2, _CQ * _NH, _DH), jnp.float32),
            pltpu.SemaphoreType.DMA,
            pltpu.SemaphoreType.DMA,
            pltpu.SemaphoreType.DMA,
            pltpu.SemaphoreType.DMA,
            pltpu.SemaphoreType.DMA,
            pltpu.SemaphoreType.DMA,
        ],
    )


def _proj_body(x_ref, w_ref, b_ref, o_ref):
    o_ref[0] = (jnp.dot(x_ref[0], w_ref[...],
                        preferred_element_type=jnp.float32) + b_ref[...])


def _out_proj(x, W_out, b_out):
    grid = (_B, _N // _NBLK)
    return pl.pallas_call(
        _proj_body,
        grid=grid,
        in_specs=[
            pl.BlockSpec((1, _NBLK, _D), lambda b, i: (b, i, 0)),
            pl.BlockSpec((_D, _D), lambda b, i: (0, 0)),
            pl.BlockSpec((1, _D), lambda b, i: (0, 0)),
        ],
        out_specs=pl.BlockSpec((1, _NBLK, _D), lambda b, i: (b, i, 0)),
        out_shape=jax.ShapeDtypeStruct((_B, _N, _D), jnp.float32),
    )(x, W_out, b_out)


def _perms():
    perm_off = np.empty(64, np.int32)
    for k in range(2):
        for p in range(_NP):
            for h in range(_NH):
                perm_off[k * 32 + p * 8 + h] = (h * _NP + p) * 2 + k
    perm_attn = np.empty(32, np.int32)
    for p in range(_NP):
        for h in range(_NH):
            perm_attn[p * 8 + h] = h * _NP + p
    perm_out = np.empty(_D, np.int32)
    for h in range(_NH):
        for i in range(_DH):
            ch = 2 * i if i < 16 else 2 * (i - 16) + 1
            perm_out[h * _DH + i] = h * _DH + ch
    return perm_off, perm_attn, perm_out


_PERM_OFF, _PERM_ATTN, _PERM_OUT = _perms()


def kernel(query, reference_pts, value_src, spatial_shape, W_off, b_off,
           W_attn, b_attn, W_val, b_val, W_out, b_out):
    del spatial_shape
    W_off_p = W_off[:, _PERM_OFF]
    b_off_p = b_off[_PERM_OFF].reshape(1, 64)
    W_attn_p = W_attn[:, _PERM_ATTN]
    b_attn_p = b_attn[_PERM_ATTN].reshape(1, 32)

    value, idx, wgt = _prep(query, reference_pts, value_src, W_val,
                            b_val.reshape(1, _D), W_off_p, b_off_p,
                            W_attn_p, b_attn_p)
    table = value.reshape(_B * _N * _NH, _DH)
    rows = _sc_gather_combine()(table, idx.reshape(_B * _N, _NC),
                                wgt.reshape(_B * _N, _NC))
    return _out_proj(rows.reshape(_B, _N, _D), W_out[_PERM_OUT, :],
                     b_out.reshape(1, _D))
